# baseline (speedup 1.0000x reference)
import functools

import jax
import jax.numpy as jnp
from jax import lax
from jax.experimental import pallas as pl
from jax.experimental.pallas import tpu as pltpu
from jax.experimental.pallas import tpu_sc as plsc

D_MODEL = 64
SCALE = 8.0
NC, NS = 2, 16
NW = NC * NS


@functools.lru_cache(maxsize=None)
def _make_lookup(n_seq, seq_len, v_rows):
    sb = n_seq // NW
    n_iter = seq_len // 2
    mesh = plsc.VectorSubcoreMesh(
        core_axis_name="c", subcore_axis_name="s", num_cores=NC, num_subcores=NS
    )

    @functools.partial(
        pl.kernel,
        out_type=jax.ShapeDtypeStruct((n_seq, seq_len, D_MODEL), jnp.float32),
        mesh=mesh,
        scratch_types=[
            pltpu.VMEM((seq_len, sb), jnp.int32),
            [pltpu.VMEM((sb, 2 * D_MODEL), jnp.float32)] * 2,
            [pltpu.VMEM((sb, D_MODEL), jnp.float32)] * 2,
            [pltpu.SemaphoreType.DMA] * 2,
            [pltpu.SemaphoreType.DMA] * 2,
        ],
        compiler_params=pltpu.CompilerParams(
            use_tc_tiling_on_sc=True, needs_layout_passes=False
        ),
    )
    def kb(xt_hbm, dup_hbm, out_hbm, idx_v, gbuf, wbuf, gsem, wsem):
        wid = lax.axis_index("s") * NC + lax.axis_index("c")
        s0 = wid * sb
        pltpu.sync_copy(xt_hbm.at[pl.ds(0, seq_len), pl.ds(s0, sb)], idx_v)

        def g_start(b, t):
            pltpu.async_copy(dup_hbm.at[idx_v.at[t]], gbuf[b], gsem[b])

        def g_wait(b):
            pltpu.make_async_copy(dup_hbm.at[idx_v.at[0]], gbuf[b], gsem[b]).wait()

        def w_start(b, t):
            pltpu.async_copy(wbuf[b], out_hbm.at[pl.ds(s0, sb), t], wsem[b])

        def w_wait(b):
            pltpu.make_async_copy(
                wbuf[b], out_hbm.at[pl.ds(s0, sb), 0], wsem[b]
            ).wait()

        for b in range(2):
            g_start(b, b)

        def step(i, carry):
            for b in range(2):
                t = 2 * i + b
                g_wait(b)

                @pl.when(i > 0)
                def _():
                    w_wait(b)

                gb, wb = gbuf[b], wbuf[b]

                @plsc.parallel_loop(0, sb, step=1, unroll=8)
                def _(j):
                    for k in range(D_MODEL // 16):
                        sl = pl.ds(16 * k, 16)
                        wb[j, sl] = gb[j, sl] * SCALE

                w_start(b, t)

                @pl.when(i < n_iter - 1)
                def _():
                    g_start(b, t + 2)

            return carry

        lax.fori_loop(0, n_iter, step, 0)
        for b in range(2):
            w_wait(b)

    return kb


def kernel(x, table):
    n_seq, seq_len = x.shape
    v_rows = table.shape[0]
    dup = jnp.concatenate(
        [table, jnp.concatenate([table[1:], table[:1]], axis=0)], axis=1
    )
    return _make_lookup(n_seq, seq_len, v_rows)(x.astype(jnp.int32).T, dup)


# speedup vs baseline: 1.2820x; 1.2820x over previous
import functools

import jax
import jax.numpy as jnp
from jax import lax
from jax.experimental import pallas as pl
from jax.experimental.pallas import tpu as pltpu
from jax.experimental.pallas import tpu_sc as plsc

D_MODEL = 64
SCALE = 8.0
NC, NS = 2, 16
NW = NC * NS


@functools.lru_cache(maxsize=None)
def _make_lookup(n_seq, seq_len):
    sb = n_seq // NW
    n_iter = seq_len // 2
    mesh = plsc.VectorSubcoreMesh(
        core_axis_name="c", subcore_axis_name="s", num_cores=NC, num_subcores=NS
    )

    @functools.partial(
        pl.kernel,
        out_type=jax.ShapeDtypeStruct((n_seq, seq_len, D_MODEL), jnp.float32),
        mesh=mesh,
        scratch_types=[
            pltpu.VMEM((seq_len, sb), jnp.int32),
            [pltpu.VMEM((sb,), jnp.int32)] * 2,
            [pltpu.VMEM((sb, 2 * D_MODEL), jnp.float32)] * 2,
            [pltpu.VMEM((sb, D_MODEL), jnp.float32)] * 2,
            [pltpu.SemaphoreType.DMA] * 2,
            [pltpu.SemaphoreType.DMA] * 2,
        ],
        compiler_params=pltpu.CompilerParams(
            use_tc_tiling_on_sc=True, needs_layout_passes=False
        ),
    )
    def kb(xt_hbm, tp_hbm, out_hbm, idx_v, pidx, gbuf, wbuf, gsem, wsem):
        wid = lax.axis_index("s") * NC + lax.axis_index("c")
        s0 = wid * sb
        pltpu.sync_copy(xt_hbm.at[pl.ds(0, seq_len), pl.ds(s0, sb)], idx_v)

        def build_pidx(b, t):
            for k in range(sb // 16):
                v = idx_v[t, pl.ds(16 * k, 16)]
                pidx[b][pl.ds(16 * k, 16)] = jax.lax.shift_right_logical(v, 1)

        def g_start(b):
            pltpu.async_copy(tp_hbm.at[pidx[b]], gbuf[b], gsem[b])

        def g_wait(b):
            pltpu.make_async_copy(tp_hbm.at[pidx[b]], gbuf[b], gsem[b]).wait()

        def w_start(b, t):
            pltpu.async_copy(wbuf[b], out_hbm.at[pl.ds(s0, sb), t], wsem[b])

        def w_wait(b):
            pltpu.make_async_copy(
                wbuf[b], out_hbm.at[pl.ds(s0, sb), 0], wsem[b]
            ).wait()

        for b in range(2):
            build_pidx(b, b)
            g_start(b)

        def step(i, carry):
            for b in range(2):
                t = 2 * i + b
                g_wait(b)

                @pl.when(i > 0)
                def _():
                    w_wait(b)

                gb, wb = gbuf[b], wbuf[b]

                @plsc.parallel_loop(0, sb // 16, step=1, unroll=2)
                def _(g):
                    hv = (idx_v[t, pl.ds(16 * g, 16)] & 1) * D_MODEL
                    for l in range(16):
                        hoff = hv[l]
                        j = 16 * g + l
                        for k in range(D_MODEL // 16):
                            sl = pl.ds(hoff + 16 * k, 16)
                            wb[j, pl.ds(16 * k, 16)] = gb[j, sl] * SCALE

                w_start(b, t)

                @pl.when(i < n_iter - 1)
                def _():
                    build_pidx(b, t + 2)
                    g_start(b)

            return carry

        lax.fori_loop(0, n_iter, step, 0)
        for b in range(2):
            w_wait(b)

    return kb


def kernel(x, table):
    n_seq, seq_len = x.shape
    tp = table.reshape(table.shape[0] // 2, 2 * D_MODEL)
    return _make_lookup(n_seq, seq_len)(x.astype(jnp.int32).T, tp)


# speedup vs baseline: 1.3162x; 1.0267x over previous
import functools

import jax
import jax.numpy as jnp
from jax import lax
from jax.experimental import pallas as pl
from jax.experimental.pallas import tpu as pltpu
from jax.experimental.pallas import tpu_sc as plsc

D_MODEL = 64
SCALE = 8.0
NC, NS = 2, 16
NW = NC * NS


@functools.lru_cache(maxsize=None)
def _make_lookup(n_seq, seq_len):
    sb = n_seq // NW
    nbuf = 4
    n_iter = seq_len // nbuf
    mesh = plsc.VectorSubcoreMesh(
        core_axis_name="c", subcore_axis_name="s", num_cores=NC, num_subcores=NS
    )

    @functools.partial(
        pl.kernel,
        out_type=jax.ShapeDtypeStruct((n_seq, seq_len, D_MODEL), jnp.float32),
        mesh=mesh,
        scratch_types=[
            pltpu.VMEM((seq_len, sb), jnp.int32),
            [pltpu.VMEM((sb,), jnp.int32)] * 4,
            [pltpu.VMEM((sb, 2 * D_MODEL), jnp.float32)] * 4,
            [pltpu.VMEM((sb, D_MODEL), jnp.float32)] * 2,
            [pltpu.SemaphoreType.DMA] * 4,
            [pltpu.SemaphoreType.DMA] * 2,
        ],
        compiler_params=pltpu.CompilerParams(
            use_tc_tiling_on_sc=True, needs_layout_passes=False
        ),
    )
    def kb(xt_hbm, tp_hbm, out_hbm, idx_v, pidx, gbuf, wbuf, gsem, wsem):
        wid = lax.axis_index("s") * NC + lax.axis_index("c")
        s0 = wid * sb
        pltpu.sync_copy(xt_hbm.at[pl.ds(0, seq_len), pl.ds(s0, sb)], idx_v)

        def build_pidx(b, t):
            for k in range(sb // 16):
                v = idx_v[t, pl.ds(16 * k, 16)]
                pidx[b][pl.ds(16 * k, 16)] = jax.lax.shift_right_logical(v, 1)

        def g_start(b):
            pltpu.async_copy(tp_hbm.at[pidx[b]], gbuf[b], gsem[b])

        def g_wait(b):
            pltpu.make_async_copy(tp_hbm.at[pidx[b]], gbuf[b], gsem[b]).wait()

        def w_start(b, t):
            pltpu.async_copy(wbuf[b % 2], out_hbm.at[pl.ds(s0, sb), t], wsem[b % 2])

        def w_wait(b):
            pltpu.make_async_copy(
                wbuf[b % 2], out_hbm.at[pl.ds(s0, sb), 0], wsem[b % 2]
            ).wait()

        for b in range(nbuf):
            build_pidx(b, b)
            g_start(b)

        def step(i, carry):
            for b in range(nbuf):
                t = nbuf * i + b
                g_wait(b)

                if b >= 2:
                    w_wait(b)
                else:

                    @pl.when(i > 0)
                    def _():
                        w_wait(b)

                gb, wb = gbuf[b], wbuf[b % 2]

                @plsc.parallel_loop(0, sb // 16, step=1, unroll=2)
                def _(g):
                    hv = (idx_v[t, pl.ds(16 * g, 16)] & 1) * D_MODEL
                    for l in range(16):
                        hoff = hv[l]
                        j = 16 * g + l
                        for k in range(D_MODEL // 16):
                            sl = pl.ds(hoff + 16 * k, 16)
                            wb[j, pl.ds(16 * k, 16)] = gb[j, sl] * SCALE

                w_start(b, t)

                @pl.when(i < n_iter - 1)
                def _():
                    build_pidx(b, t + nbuf)
                    g_start(b)

            return carry

        lax.fori_loop(0, n_iter, step, 0)
        for b in range(2):
            w_wait(b)

    return kb


def kernel(x, table):
    n_seq, seq_len = x.shape
    tp = table.reshape(table.shape[0] // 2, 2 * D_MODEL)
    return _make_lookup(n_seq, seq_len)(x.astype(jnp.int32).T, tp)
